# baseline (speedup 1.0000x reference)
import functools

import jax
import jax.numpy as jnp
from jax import lax
from jax.experimental import layout as jax_layout
from jax.experimental import pallas as pl
from jax.experimental.pallas import tpu as pltpu
from jax.experimental.pallas import tpu_sc as plsc

B, L, H = 4096, 20, 8
VOCAB, EMBED = 1000, 64
NC, NS = 2, 16
NW = NC * NS
BPW = B // NW
NPAIR = EMBED // 2
TABLE_WORDS = VOCAB * NPAIR


def _sc_kernel(table_hbm, idx_hbm, out_hbm, table_v,
               idx_v0, idx_v1, out_v0, out_v1,
               sem_i0, sem_i1, sem_o0, sem_o1):
  wid = lax.axis_index("s") * NC + lax.axis_index("c")
  pltpu.sync_copy(table_hbm, table_v)
  b0 = wid * BPW
  idx_bufs = (idx_v0, idx_v1)
  out_bufs = (out_v0, out_v1)
  sem_i = (sem_i0, sem_i1)
  sem_o = (sem_o0, sem_o1)

  def compute(idx_v, out_v):
    def grp_body(g, carry2):
      hv = [idx_v[h, pl.ds(g * 16, 16)] * NPAIR for h in range(H)]
      for j in range(16):
        rows0 = []
        rows1 = []
        for h in range(H):
          base = hv[h][j]
          rows0.append(
              plsc.bitcast(table_v[pl.ds(base, 16)], jnp.bfloat16))
          rows1.append(
              plsc.bitcast(table_v[pl.ds(base + 16, 16)], jnp.bfloat16))
        acc0 = ((rows0[0] + rows0[1]) + (rows0[2] + rows0[3])) + (
            (rows0[4] + rows0[5]) + (rows0[6] + rows0[7]))
        acc1 = ((rows1[0] + rows1[1]) + (rows1[2] + rows1[3])) + (
            (rows1[4] + rows1[5]) + (rows1[6] + rows1[7]))
        lo0, hi0 = plsc.unpack(acc0, format=plsc.PackFormat.INTERLEAVED)
        lo1, hi1 = plsc.unpack(acc1, format=plsc.PackFormat.INTERLEAVED)
        bag = g * 16 + j
        out_v[bag, pl.ds(0, 16)] = lo0
        out_v[bag, pl.ds(16, 16)] = lo1
        out_v[bag, pl.ds(32, 16)] = hi0
        out_v[bag, pl.ds(48, 16)] = hi1
      return carry2

    lax.fori_loop(0, BPW // 16, grp_body, 0, unroll=2)

  pltpu.async_copy(idx_hbm.at[0, :, pl.ds(b0, BPW)], idx_v0, sem_i0)
  pltpu.async_copy(idx_hbm.at[1, :, pl.ds(b0, BPW)], idx_v1, sem_i1)

  def pair_body(g, carry):
    for s in range(2):
      l = 2 * g + s
      pltpu.make_async_copy(
          idx_hbm.at[l, :, pl.ds(b0, BPW)], idx_bufs[s], sem_i[s]).wait()

      @pl.when(g > 0)
      def _():
        pltpu.make_async_copy(
            out_bufs[s], out_hbm.at[l - 2, pl.ds(b0, BPW), :],
            sem_o[s]).wait()

      compute(idx_bufs[s], out_bufs[s])

      @pl.when(g < L // 2 - 1)
      def _():
        pltpu.async_copy(
            idx_hbm.at[l + 2, :, pl.ds(b0, BPW)], idx_bufs[s], sem_i[s])

      pltpu.async_copy(
          out_bufs[s], out_hbm.at[l, pl.ds(b0, BPW), :], sem_o[s])
    return carry

  lax.fori_loop(0, L // 2, pair_body, 0, unroll=False)
  for s in range(2):
    pltpu.make_async_copy(
        out_bufs[s], out_hbm.at[L - 2 + s, pl.ds(b0, BPW), :],
        sem_o[s]).wait()


def _kernel_impl(content_input, W, b):
  idx_t = jnp.transpose(content_input.astype(jnp.int32), (1, 2, 0))
  wp = (W + b[:, None] * (1.0 / H)).astype(jnp.bfloat16)
  packed = lax.bitcast_convert_type(
      jnp.stack([wp[:NPAIR], wp[NPAIR:]], axis=-1), jnp.int32)
  table = packed.T.reshape(-1)
  run = pl.kernel(
      _sc_kernel,
      out_type=jax.ShapeDtypeStruct((L, B, EMBED), jnp.float32),
      mesh=plsc.VectorSubcoreMesh(
          core_axis_name="c", subcore_axis_name="s", num_cores=NC,
          num_subcores=NS),
      scratch_types=[
          pltpu.VMEM((TABLE_WORDS,), jnp.int32),
          pltpu.VMEM((H, BPW), jnp.int32),
          pltpu.VMEM((H, BPW), jnp.int32),
          pltpu.VMEM((BPW, EMBED), jnp.float32),
          pltpu.VMEM((BPW, EMBED), jnp.float32),
          pltpu.SemaphoreType.DMA,
          pltpu.SemaphoreType.DMA,
          pltpu.SemaphoreType.DMA,
          pltpu.SemaphoreType.DMA,
      ],
      compiler_params=pltpu.CompilerParams(needs_layout_passes=False),
  )
  return jnp.transpose(run(table, idx_t), (1, 0, 2))


@functools.lru_cache(maxsize=None)
def _jitted(dev):
  fmt = jax_layout.Format(
      jax_layout.Layout(major_to_minor=(1, 0, 2), tiling=((1,),)),
      jax.sharding.SingleDeviceSharding(dev))
  return jax.jit(_kernel_impl, out_shardings=fmt)


def kernel(content_input, W, b):
  if isinstance(content_input, jax.core.Tracer):
    return _kernel_impl(content_input, W, b)
  dev = next(iter(content_input.devices()))
  return _jitted(dev)(content_input, W, b)


# speedup vs baseline: 1.1489x; 1.1489x over previous
import functools

import jax
import jax.numpy as jnp
from jax import lax
from jax.experimental import layout as jax_layout
from jax.experimental import pallas as pl
from jax.experimental.pallas import tpu as pltpu
from jax.experimental.pallas import tpu_sc as plsc

B, L, H = 4096, 20, 8
VOCAB, EMBED = 1000, 64
NC, NS = 2, 16
NW = NC * NS
BPW = B // NW
NPAIR = EMBED // 2
TABLE_WORDS = VOCAB * NPAIR


def _sc_kernel(table_hbm, idx_hbm, out_hbm, table_v,
               idx_v0, idx_v1, out_v0, out_v1,
               sem_i0, sem_i1, sem_o0, sem_o1):
  wid = lax.axis_index("s") * NC + lax.axis_index("c")
  pltpu.sync_copy(table_hbm, table_v)
  b0 = wid * BPW
  idx_bufs = (idx_v0, idx_v1)
  out_bufs = (out_v0, out_v1)
  sem_i = (sem_i0, sem_i1)
  sem_o = (sem_o0, sem_o1)

  def compute(idx_v, out_v):
    def grp_body(g, carry2):
      hv = [idx_v[h, pl.ds(g * 16, 16)] * NPAIR for h in range(H)]
      for j in range(16):
        rows0 = []
        rows1 = []
        for h in range(H):
          base = hv[h][j]
          rows0.append(
              plsc.bitcast(table_v[pl.ds(base, 16)], jnp.bfloat16))
          rows1.append(
              plsc.bitcast(table_v[pl.ds(base + 16, 16)], jnp.bfloat16))
        acc0 = ((rows0[0] + rows0[1]) + (rows0[2] + rows0[3])) + (
            (rows0[4] + rows0[5]) + (rows0[6] + rows0[7]))
        acc1 = ((rows1[0] + rows1[1]) + (rows1[2] + rows1[3])) + (
            (rows1[4] + rows1[5]) + (rows1[6] + rows1[7]))
        lo0, hi0 = plsc.unpack(acc0, format=plsc.PackFormat.INTERLEAVED)
        lo1, hi1 = plsc.unpack(acc1, format=plsc.PackFormat.INTERLEAVED)
        bag = g * 16 + j
        out_v[bag, pl.ds(0, 16)] = lo0
        out_v[bag, pl.ds(16, 16)] = lo1
        out_v[bag, pl.ds(32, 16)] = hi0
        out_v[bag, pl.ds(48, 16)] = hi1
      return carry2

    lax.fori_loop(0, BPW // 16, grp_body, 0, unroll=False)

  pltpu.async_copy(idx_hbm.at[0, :, pl.ds(b0, BPW)], idx_v0, sem_i0)
  pltpu.async_copy(idx_hbm.at[1, :, pl.ds(b0, BPW)], idx_v1, sem_i1)

  def pair_body(g, carry):
    for s in range(2):
      l = 2 * g + s
      pltpu.make_async_copy(
          idx_hbm.at[l, :, pl.ds(b0, BPW)], idx_bufs[s], sem_i[s]).wait()

      @pl.when(g > 0)
      def _():
        pltpu.make_async_copy(
            out_bufs[s], out_hbm.at[l - 2, pl.ds(b0, BPW), :],
            sem_o[s]).wait()

      compute(idx_bufs[s], out_bufs[s])

      @pl.when(g < L // 2 - 1)
      def _():
        pltpu.async_copy(
            idx_hbm.at[l + 2, :, pl.ds(b0, BPW)], idx_bufs[s], sem_i[s])

      pltpu.async_copy(
          out_bufs[s], out_hbm.at[l, pl.ds(b0, BPW), :], sem_o[s])
    return carry

  lax.fori_loop(0, L // 2, pair_body, 0, unroll=False)
  for s in range(2):
    pltpu.make_async_copy(
        out_bufs[s], out_hbm.at[L - 2 + s, pl.ds(b0, BPW), :],
        sem_o[s]).wait()


def _kernel_impl(content_input, W, b):
  idx_t = jnp.transpose(content_input.astype(jnp.int32), (1, 2, 0))
  wp = (W + b[:, None] * (1.0 / H)).astype(jnp.bfloat16)
  packed = lax.bitcast_convert_type(
      jnp.stack([wp[:NPAIR], wp[NPAIR:]], axis=-1), jnp.int32)
  table = packed.T.reshape(-1)
  run = pl.kernel(
      _sc_kernel,
      out_type=jax.ShapeDtypeStruct((L, B, EMBED), jnp.float32),
      mesh=plsc.VectorSubcoreMesh(
          core_axis_name="c", subcore_axis_name="s", num_cores=NC,
          num_subcores=NS),
      scratch_types=[
          pltpu.VMEM((TABLE_WORDS,), jnp.int32),
          pltpu.VMEM((H, BPW), jnp.int32),
          pltpu.VMEM((H, BPW), jnp.int32),
          pltpu.VMEM((BPW, EMBED), jnp.float32),
          pltpu.VMEM((BPW, EMBED), jnp.float32),
          pltpu.SemaphoreType.DMA,
          pltpu.SemaphoreType.DMA,
          pltpu.SemaphoreType.DMA,
          pltpu.SemaphoreType.DMA,
      ],
      compiler_params=pltpu.CompilerParams(needs_layout_passes=False),
  )
  return jnp.transpose(run(table, idx_t), (1, 0, 2))


@functools.lru_cache(maxsize=None)
def _jitted(dev):
  fmt = jax_layout.Format(
      jax_layout.Layout(major_to_minor=(1, 0, 2), tiling=((1,),)),
      jax.sharding.SingleDeviceSharding(dev))
  return jax.jit(_kernel_impl, out_shardings=fmt)


def kernel(content_input, W, b):
  if isinstance(content_input, jax.core.Tracer):
    return _kernel_impl(content_input, W, b)
  dev = next(iter(content_input.devices()))
  return _jitted(dev)(content_input, W, b)
